# baseline (device time: 35003 ns/iter reference)
import jax
import jax.numpy as jnp
from jax import lax
from jax.experimental import pallas as pl
from jax.experimental.pallas import tpu as pltpu

N_DEV = 16
T = 32
D = 512
F = 1024
EPD = 2
E = N_DEV * EPD
CE = 10


def kernel(x, router, W1, W2):
    r_t_shard = router.T

    def body(x_ref, rt_ref, w1_hbm, w2_hbm, out_ref,
             w1_ref, w2_ref, rbuf, sbuf, recvbuf, ycomb, cbuf,
             wsem, rsend, rrecv, dsend, drecv, csend, crecv):
        my_i = lax.axis_index("i")

        w1_copy = pltpu.make_async_copy(w1_hbm, w1_ref, wsem.at[0])
        w2_copy = pltpu.make_async_copy(w2_hbm, w2_ref, wsem.at[1])
        w1_copy.start()
        w2_copy.start()

        rbuf[my_i] = rt_ref[...]
        for p in range(N_DEV):
            @pl.when(p != my_i)
            def _():
                pltpu.make_async_remote_copy(
                    src_ref=rt_ref,
                    dst_ref=rbuf.at[my_i],
                    send_sem=rsend.at[p],
                    recv_sem=rrecv.at[my_i],
                    device_id=(p,),
                    device_id_type=pl.DeviceIdType.MESH,
                ).start()
        for p in range(N_DEV):
            @pl.when(p != my_i)
            def _():
                pltpu.make_async_remote_copy(
                    src_ref=rt_ref,
                    dst_ref=rbuf.at[p],
                    send_sem=rsend.at[p],
                    recv_sem=rrecv.at[p],
                    device_id=(p,),
                    device_id_type=pl.DeviceIdType.MESH,
                ).wait_recv()
        for p in range(N_DEV):
            @pl.when(p != my_i)
            def _():
                pltpu.make_async_remote_copy(
                    src_ref=rt_ref,
                    dst_ref=rbuf.at[my_i],
                    send_sem=rsend.at[p],
                    recv_sem=rrecv.at[my_i],
                    device_id=(p,),
                    device_id_type=pl.DeviceIdType.MESH,
                ).wait_send()

        xv = x_ref[...]
        r_t = rbuf[...].reshape(E, D)
        gates_t = lax.dot_general(
            r_t, xv, (((1,), (1,)), ((), ())),
            precision=lax.Precision.HIGHEST,
            preferred_element_type=jnp.float32,
        )
        m1 = jnp.max(gates_t, axis=0)[None, :]
        i1 = jnp.argmax(gates_t, axis=0)[None, :]
        rows = lax.broadcasted_iota(jnp.int32, (E, T), 0)
        masked = jnp.where(rows == i1, -1e30, gates_t)
        m2 = jnp.max(masked, axis=0)[None, :]
        i2 = jnp.argmax(masked, axis=0)[None, :]
        ex = jnp.exp(m2 - m1)
        w1g = 1.0 / (1.0 + ex)
        w2g = ex / (1.0 + ex)

        exp_e = jnp.concatenate([i1, i2], axis=1)
        w_e = jnp.concatenate([w1g, w2g], axis=1)
        match = (lax.broadcasted_iota(jnp.int32, (E, 2 * T), 0)
                 == exp_e).astype(jnp.float32)
        tri_a = lax.broadcasted_iota(jnp.int32, (2 * T, 2 * T), 0)
        tri_b = lax.broadcasted_iota(jnp.int32, (2 * T, 2 * T), 1)
        tri = (tri_a < tri_b).astype(jnp.float32)
        rank = lax.dot_general(
            match, tri, (((1,), (0,)), ((), ())),
            preferred_element_type=jnp.float32,
        )
        c_iota = lax.broadcasted_iota(jnp.int32, (E, CE, 2 * T), 1)
        p3 = ((rank[:, None, :] == c_iota.astype(jnp.float32))
              & (match[:, None, :] == 1.0))
        pm = p3.astype(jnp.float32).reshape(E * CE, 2 * T)
        pw = pm * w_e

        xe = jnp.concatenate([xv, xv], axis=0)
        send_rows = lax.dot_general(
            pw, xe, (((1,), (0,)), ((), ())),
            preferred_element_type=jnp.float32,
        )
        sbuf[...] = send_rows.reshape(N_DEV, EPD * CE, D).astype(jnp.bfloat16)

        recvbuf[my_i] = sbuf[my_i]
        for p in range(N_DEV):
            @pl.when(p != my_i)
            def _():
                pltpu.make_async_remote_copy(
                    src_ref=sbuf.at[p],
                    dst_ref=recvbuf.at[my_i],
                    send_sem=dsend.at[p],
                    recv_sem=drecv.at[my_i],
                    device_id=(p,),
                    device_id_type=pl.DeviceIdType.MESH,
                ).start()
        for p in range(N_DEV):
            @pl.when(p != my_i)
            def _():
                pltpu.make_async_remote_copy(
                    src_ref=sbuf.at[p],
                    dst_ref=recvbuf.at[p],
                    send_sem=dsend.at[p],
                    recv_sem=drecv.at[p],
                    device_id=(p,),
                    device_id_type=pl.DeviceIdType.MESH,
                ).wait_recv()
        for p in range(N_DEV):
            @pl.when(p != my_i)
            def _():
                pltpu.make_async_remote_copy(
                    src_ref=sbuf.at[p],
                    dst_ref=recvbuf.at[my_i],
                    send_sem=dsend.at[p],
                    recv_sem=drecv.at[my_i],
                    device_id=(p,),
                    device_id_type=pl.DeviceIdType.MESH,
                ).wait_send()

        w1_copy.wait()
        w2_copy.wait()
        xr = recvbuf[...].astype(jnp.float32)
        for l in range(EPD):
            xl = xr[:, l * CE:(l + 1) * CE, :]
            h = lax.dot_general(
                xl, w1_ref[l], (((2,), (0,)), ((), ())),
                preferred_element_type=jnp.float32,
            )
            h = jnp.maximum(h, 0.0)
            y = lax.dot_general(
                h, w2_ref[l], (((2,), (0,)), ((), ())),
                preferred_element_type=jnp.float32,
            )
            ycomb[:, l * CE:(l + 1) * CE, :] = y.astype(jnp.bfloat16)

        cbuf[my_i] = ycomb[my_i]
        for p in range(N_DEV):
            @pl.when(p != my_i)
            def _():
                pltpu.make_async_remote_copy(
                    src_ref=ycomb.at[p],
                    dst_ref=cbuf.at[my_i],
                    send_sem=csend.at[p],
                    recv_sem=crecv.at[my_i],
                    device_id=(p,),
                    device_id_type=pl.DeviceIdType.MESH,
                ).start()
        for p in range(N_DEV):
            @pl.when(p != my_i)
            def _():
                pltpu.make_async_remote_copy(
                    src_ref=ycomb.at[p],
                    dst_ref=cbuf.at[p],
                    send_sem=csend.at[p],
                    recv_sem=crecv.at[p],
                    device_id=(p,),
                    device_id_type=pl.DeviceIdType.MESH,
                ).wait_recv()
        for p in range(N_DEV):
            @pl.when(p != my_i)
            def _():
                pltpu.make_async_remote_copy(
                    src_ref=ycomb.at[p],
                    dst_ref=cbuf.at[my_i],
                    send_sem=csend.at[p],
                    recv_sem=crecv.at[my_i],
                    device_id=(p,),
                    device_id_type=pl.DeviceIdType.MESH,
                ).wait_send()

        yall = cbuf[...].reshape(E * CE, D).astype(jnp.float32)
        acc = lax.dot_general(
            pm, yall, (((0,), (0,)), ((), ())),
            preferred_element_type=jnp.float32,
        )
        out_ref[...] = acc[:T, :] + acc[T:, :]

    return pl.pallas_call(
        body,
        out_shape=jax.ShapeDtypeStruct((T, D), jnp.float32),
        in_specs=[
            pl.BlockSpec(memory_space=pltpu.VMEM),
            pl.BlockSpec(memory_space=pltpu.VMEM),
            pl.BlockSpec(memory_space=pltpu.MemorySpace.HBM),
            pl.BlockSpec(memory_space=pltpu.MemorySpace.HBM),
        ],
        out_specs=pl.BlockSpec(memory_space=pltpu.VMEM),
        scratch_shapes=[
            pltpu.VMEM((EPD, D, F), jnp.float32),
            pltpu.VMEM((EPD, F, D), jnp.float32),
            pltpu.VMEM((N_DEV, EPD, D), jnp.float32),
            pltpu.VMEM((N_DEV, EPD * CE, D), jnp.bfloat16),
            pltpu.VMEM((N_DEV, EPD * CE, D), jnp.bfloat16),
            pltpu.VMEM((N_DEV, EPD * CE, D), jnp.bfloat16),
            pltpu.VMEM((N_DEV, EPD * CE, D), jnp.bfloat16),
            pltpu.SemaphoreType.DMA((2,)),
            pltpu.SemaphoreType.DMA((N_DEV,)),
            pltpu.SemaphoreType.DMA((N_DEV,)),
            pltpu.SemaphoreType.DMA((N_DEV,)),
            pltpu.SemaphoreType.DMA((N_DEV,)),
            pltpu.SemaphoreType.DMA((N_DEV,)),
            pltpu.SemaphoreType.DMA((N_DEV,)),
        ],
        compiler_params=pltpu.CompilerParams(has_side_effects=True),
    )(x, r_t_shard, W1, W2)


# device time: 34459 ns/iter; 1.0158x vs baseline; 1.0158x over previous
import jax
import jax.numpy as jnp
from jax import lax
from jax.experimental import pallas as pl
from jax.experimental.pallas import tpu as pltpu

N_DEV = 16
T = 32
D = 512
F = 1024
EPD = 2
E = N_DEV * EPD
CE = 10


def kernel(x, router, W1, W2):
    r_t_shard = router.T

    def body(x_ref, rt_ref, w1_hbm, w2_hbm, out_ref,
             w1_ref, w2_ref, rbuf, sbuf, recvbuf, ycomb, cbuf,
             wsem, rsend, rrecv, dsend, drecv, csend, crecv):
        my_i = lax.axis_index("i")

        w1_copy = pltpu.make_async_copy(w1_hbm, w1_ref, wsem.at[0])
        w2_copy = pltpu.make_async_copy(w2_hbm, w2_ref, wsem.at[1])
        w1_copy.start()
        w2_copy.start()

        rbuf[my_i] = rt_ref[...]
        for p in range(N_DEV):
            @pl.when(p != my_i)
            def _():
                pltpu.make_async_remote_copy(
                    src_ref=rt_ref,
                    dst_ref=rbuf.at[my_i],
                    send_sem=rsend.at[p],
                    recv_sem=rrecv.at[my_i],
                    device_id=(p,),
                    device_id_type=pl.DeviceIdType.MESH,
                ).start()
        for p in range(N_DEV):
            @pl.when(p != my_i)
            def _():
                pltpu.make_async_remote_copy(
                    src_ref=rt_ref,
                    dst_ref=rbuf.at[p],
                    send_sem=rsend.at[p],
                    recv_sem=rrecv.at[p],
                    device_id=(p,),
                    device_id_type=pl.DeviceIdType.MESH,
                ).wait_recv()
        for p in range(N_DEV):
            @pl.when(p != my_i)
            def _():
                pltpu.make_async_remote_copy(
                    src_ref=rt_ref,
                    dst_ref=rbuf.at[my_i],
                    send_sem=rsend.at[p],
                    recv_sem=rrecv.at[my_i],
                    device_id=(p,),
                    device_id_type=pl.DeviceIdType.MESH,
                ).wait_send()

        xv = x_ref[...]
        r_t = rbuf[...].reshape(E, D)
        gates_t = lax.dot_general(
            r_t, xv, (((1,), (1,)), ((), ())),
            precision=lax.Precision.HIGHEST,
            preferred_element_type=jnp.float32,
        )
        m1 = jnp.max(gates_t, axis=0)[None, :]
        i1 = jnp.argmax(gates_t, axis=0)[None, :]
        rows = lax.broadcasted_iota(jnp.int32, (E, T), 0)
        masked = jnp.where(rows == i1, -1e30, gates_t)
        m2 = jnp.max(masked, axis=0)[None, :]
        i2 = jnp.argmax(masked, axis=0)[None, :]
        ex = jnp.exp(m2 - m1)
        w1g = 1.0 / (1.0 + ex)
        w2g = ex / (1.0 + ex)

        exp_e = jnp.concatenate([i1, i2], axis=1)
        w_e = jnp.concatenate([w1g, w2g], axis=1)
        match = (lax.broadcasted_iota(jnp.int32, (E, 2 * T), 0)
                 == exp_e).astype(jnp.float32)
        tri_a = lax.broadcasted_iota(jnp.int32, (2 * T, 2 * T), 0)
        tri_b = lax.broadcasted_iota(jnp.int32, (2 * T, 2 * T), 1)
        tri = (tri_a < tri_b).astype(jnp.float32)
        rank = lax.dot_general(
            match, tri, (((1,), (0,)), ((), ())),
            preferred_element_type=jnp.float32,
        )
        c_iota = lax.broadcasted_iota(jnp.int32, (E, CE, 2 * T), 1)
        p3 = ((rank[:, None, :] == c_iota.astype(jnp.float32))
              & (match[:, None, :] == 1.0))
        pm = p3.astype(jnp.float32).reshape(E * CE, 2 * T)
        pw = pm * w_e

        xe = jnp.concatenate([xv, xv], axis=0)
        send_rows = lax.dot_general(
            pw, xe, (((1,), (0,)), ((), ())),
            preferred_element_type=jnp.float32,
        )
        sbuf[...] = send_rows.reshape(N_DEV, EPD * CE, D).astype(jnp.bfloat16)

        recvbuf[my_i] = sbuf[my_i]
        for p in range(N_DEV):
            @pl.when(p != my_i)
            def _():
                pltpu.make_async_remote_copy(
                    src_ref=sbuf.at[p],
                    dst_ref=recvbuf.at[my_i],
                    send_sem=dsend.at[p],
                    recv_sem=drecv.at[my_i],
                    device_id=(p,),
                    device_id_type=pl.DeviceIdType.MESH,
                ).start()

        GRP = 4
        for g in range(N_DEV // GRP):
            lo = g * GRP
            for p in range(lo, lo + GRP):
                @pl.when(p != my_i)
                def _():
                    pltpu.make_async_remote_copy(
                        src_ref=sbuf.at[p],
                        dst_ref=recvbuf.at[p],
                        send_sem=dsend.at[p],
                        recv_sem=drecv.at[p],
                        device_id=(p,),
                        device_id_type=pl.DeviceIdType.MESH,
                    ).wait_recv()
            if g == 0:
                w1_copy.wait()
                w2_copy.wait()
            xg = recvbuf[lo:lo + GRP].astype(jnp.float32)
            for l in range(EPD):
                xl = xg[:, l * CE:(l + 1) * CE, :]
                h = lax.dot_general(
                    xl, w1_ref[l], (((2,), (0,)), ((), ())),
                    preferred_element_type=jnp.float32,
                )
                h = jnp.maximum(h, 0.0)
                y = lax.dot_general(
                    h, w2_ref[l], (((2,), (0,)), ((), ())),
                    preferred_element_type=jnp.float32,
                )
                ycomb[lo:lo + GRP, l * CE:(l + 1) * CE, :] = (
                    y.astype(jnp.bfloat16))
            for p in range(lo, lo + GRP):
                @pl.when(p == my_i)
                def _():
                    cbuf[my_i] = ycomb[my_i]
                @pl.when(p != my_i)
                def _():
                    pltpu.make_async_remote_copy(
                        src_ref=ycomb.at[p],
                        dst_ref=cbuf.at[my_i],
                        send_sem=csend.at[p],
                        recv_sem=crecv.at[my_i],
                        device_id=(p,),
                        device_id_type=pl.DeviceIdType.MESH,
                    ).start()

        for p in range(N_DEV):
            @pl.when(p != my_i)
            def _():
                pltpu.make_async_remote_copy(
                    src_ref=ycomb.at[p],
                    dst_ref=cbuf.at[p],
                    send_sem=csend.at[p],
                    recv_sem=crecv.at[p],
                    device_id=(p,),
                    device_id_type=pl.DeviceIdType.MESH,
                ).wait_recv()
        for p in range(N_DEV):
            @pl.when(p != my_i)
            def _():
                pltpu.make_async_remote_copy(
                    src_ref=sbuf.at[p],
                    dst_ref=recvbuf.at[my_i],
                    send_sem=dsend.at[p],
                    recv_sem=drecv.at[my_i],
                    device_id=(p,),
                    device_id_type=pl.DeviceIdType.MESH,
                ).wait_send()
        for p in range(N_DEV):
            @pl.when(p != my_i)
            def _():
                pltpu.make_async_remote_copy(
                    src_ref=ycomb.at[p],
                    dst_ref=cbuf.at[my_i],
                    send_sem=csend.at[p],
                    recv_sem=crecv.at[my_i],
                    device_id=(p,),
                    device_id_type=pl.DeviceIdType.MESH,
                ).wait_send()

        yall = cbuf[...].reshape(E * CE, D).astype(jnp.float32)
        acc = lax.dot_general(
            pm, yall, (((0,), (0,)), ((), ())),
            preferred_element_type=jnp.float32,
        )
        out_ref[...] = acc[:T, :] + acc[T:, :]

    return pl.pallas_call(
        body,
        out_shape=jax.ShapeDtypeStruct((T, D), jnp.float32),
        in_specs=[
            pl.BlockSpec(memory_space=pltpu.VMEM),
            pl.BlockSpec(memory_space=pltpu.VMEM),
            pl.BlockSpec(memory_space=pltpu.MemorySpace.HBM),
            pl.BlockSpec(memory_space=pltpu.MemorySpace.HBM),
        ],
        out_specs=pl.BlockSpec(memory_space=pltpu.VMEM),
        scratch_shapes=[
            pltpu.VMEM((EPD, D, F), jnp.float32),
            pltpu.VMEM((EPD, F, D), jnp.float32),
            pltpu.VMEM((N_DEV, EPD, D), jnp.float32),
            pltpu.VMEM((N_DEV, EPD * CE, D), jnp.bfloat16),
            pltpu.VMEM((N_DEV, EPD * CE, D), jnp.bfloat16),
            pltpu.VMEM((N_DEV, EPD * CE, D), jnp.bfloat16),
            pltpu.VMEM((N_DEV, EPD * CE, D), jnp.bfloat16),
            pltpu.SemaphoreType.DMA((2,)),
            pltpu.SemaphoreType.DMA((N_DEV,)),
            pltpu.SemaphoreType.DMA((N_DEV,)),
            pltpu.SemaphoreType.DMA((N_DEV,)),
            pltpu.SemaphoreType.DMA((N_DEV,)),
            pltpu.SemaphoreType.DMA((N_DEV,)),
            pltpu.SemaphoreType.DMA((N_DEV,)),
        ],
        compiler_params=pltpu.CompilerParams(has_side_effects=True),
    )(x, r_t_shard, W1, W2)
